# Initial kernel scaffold; baseline (speedup 1.0000x reference)
import jax
import jax.numpy as jnp
from jax.experimental import pallas as pl


def kernel(x, edge_index, edge_weight, W1, b1, W2, b2):
    raise NotImplementedError("write your pallas kernel here")


# speedup vs baseline: 6.1679x; 6.1679x over previous
import jax
import jax.numpy as jnp
from jax import lax
from jax.experimental import pallas as pl
from jax.experimental.pallas import tpu as pltpu
from jax.experimental.pallas import tpu_sc as plsc

N = 10000
E = 160000
NFEAT = 256
NHID = 512
NCLASS = 32

NPAD = 10240
EP = 163840
EROWS = EP // 128
NSUB = 16
NCORE = 2

_f32 = jnp.float32
_i32 = jnp.int32


def _deg_body(dst_hbm, ew_hbm, out_hbm, dst_v, ew_v, rows_v, zbuf, bounce, deg_sp):
    c = lax.axis_index("c")
    s = lax.axis_index("s")
    wid = c * NSUB + s
    rpt = EROWS // (NCORE * NSUB)
    base_n = s * (NPAD // NSUB)

    zv = jnp.zeros((16,), _f32)
    for r in range(16):
        zbuf[r, :] = zv

    @pl.loop(0, (NPAD // NSUB) // 16)
    def _(i):
        pltpu.sync_copy(zbuf, deg_sp.at[pl.ds(base_n + i * 16, 16)])

    pltpu.sync_copy(dst_hbm.at[pl.ds(wid * rpt, rpt)], dst_v)
    pltpu.sync_copy(ew_hbm.at[pl.ds(wid * rpt, rpt)], ew_v)
    plsc.subcore_barrier()

    @pl.loop(0, rpt)
    def _(g):
        @pl.loop(0, 8)
        def _(q):
            wv = ew_v[g, pl.ds(q * 16, 16)]
            for l in range(16):
                rows_v[q * 16 + l, :] = jnp.full((16,), wv[l], _f32)

        pltpu.sync_copy(rows_v, deg_sp.at[dst_v.at[g]], add=True)

    plsc.subcore_barrier()

    @pl.loop(0, 5)
    def _(i):
        off = base_n + i * 128
        pltpu.sync_copy(deg_sp.at[pl.ds(off, 128)], bounce)
        pltpu.sync_copy(bounce, out_hbm.at[c, pl.ds(off, 128)])


def _make_agg_body(width, feature_split):
    nvec = width // 16
    rpt = EROWS // NSUB if feature_split else EROWS // (NCORE * NSUB)
    nblk = rpt // 8

    in_place = width == 128

    def body(t0_hbm, src_hbm, dst_hbm, ew_hbm, dinv_hbm, out_hbm,
             src_v, dst_v, ew_v, rows_v, out_rows_v, dsrc_v, ddst_v, norm_v,
             zbuf, agg_sp):
        c = lax.axis_index("c")
        s = lax.axis_index("s")
        row0 = (s * rpt) if feature_split else ((c * NSUB + s) * rpt)
        base_n = s * (NPAD // NSUB)

        zv = jnp.zeros((16,), _f32)
        for r in range(16):
            for f in range(nvec):
                zbuf[r, pl.ds(f * 16, 16)] = zv

        @pl.loop(0, (NPAD // NSUB) // 16)
        def _(i):
            pltpu.sync_copy(zbuf, agg_sp.at[pl.ds(base_n + i * 16, 16)])

        plsc.subcore_barrier()

        @pl.loop(0, nblk)
        def _(b):
            pltpu.sync_copy(src_hbm.at[pl.ds(row0 + b * 8, 8)], src_v)
            pltpu.sync_copy(dst_hbm.at[pl.ds(row0 + b * 8, 8)], dst_v)
            pltpu.sync_copy(ew_hbm.at[pl.ds(row0 + b * 8, 8)], ew_v)

            @pl.loop(0, 8)
            def _(g):
                pltpu.sync_copy(t0_hbm.at[src_v.at[g]], rows_v)
                pltpu.sync_copy(dinv_hbm.at[src_v.at[g]], dsrc_v)
                pltpu.sync_copy(dinv_hbm.at[dst_v.at[g]], ddst_v)

                for q in range(8):
                    sl = pl.ds(q * 16, 16)
                    norm_v[sl] = dsrc_v[sl] * ew_v[g, sl] * ddst_v[sl]

                @pl.loop(0, 8)
                def _(q):
                    nv = norm_v[pl.ds(q * 16, 16)]
                    for l in range(16):
                        sc = nv[l]
                        for f in range(nvec):
                            fs = pl.ds(f * 16, 16)
                            out_rows_v[q * 16 + l, fs] = rows_v[q * 16 + l, fs] * sc

                pltpu.sync_copy(out_rows_v, agg_sp.at[dst_v.at[g]], add=True)

        plsc.subcore_barrier()

        @pl.loop(0, 5)
        def _(i):
            off = base_n + i * 128
            pltpu.sync_copy(agg_sp.at[pl.ds(off, 128)], out_rows_v)
            pltpu.sync_copy(out_rows_v, out_hbm.at[c, pl.ds(off, 128)])

    def body_in_place(t_hbm, src_hbm, dst_hbm, ew_hbm, dinv_hbm, out_hbm,
                      src_v, dst_v, ew_v, idxa_v, rows_v, dsrc_v, ddst_v, norm_v,
                      zbuf, agg_sp):
        c = lax.axis_index("c")
        s = lax.axis_index("s")
        row0 = (s * rpt) if feature_split else ((c * NSUB + s) * rpt)
        base_n = s * (NPAD // NSUB)

        zv = jnp.zeros((16,), _f32)
        for r in range(16):
            for f in range(nvec):
                zbuf[r, pl.ds(f * 16, 16)] = zv

        @pl.loop(0, (NPAD // NSUB) // 16)
        def _(i):
            pltpu.sync_copy(zbuf, agg_sp.at[pl.ds(base_n + i * 16, 16)])

        plsc.subcore_barrier()

        @pl.loop(0, nblk)
        def _(b):
            pltpu.sync_copy(src_hbm.at[pl.ds(row0 + b * 8, 8)], src_v)
            pltpu.sync_copy(dst_hbm.at[pl.ds(row0 + b * 8, 8)], dst_v)
            pltpu.sync_copy(ew_hbm.at[pl.ds(row0 + b * 8, 8)], ew_v)

            @pl.loop(0, 8)
            def _(g):
                coff = c * N
                for q in range(8):
                    sl = pl.ds(q * 16, 16)
                    idxa_v[sl] = src_v[g, sl] + coff
                pltpu.sync_copy(t_hbm.at[idxa_v], rows_v)
                pltpu.sync_copy(dinv_hbm.at[src_v.at[g]], dsrc_v)
                pltpu.sync_copy(dinv_hbm.at[dst_v.at[g]], ddst_v)

                for q in range(8):
                    sl = pl.ds(q * 16, 16)
                    norm_v[sl] = dsrc_v[sl] * ew_v[g, sl] * ddst_v[sl]

                @pl.loop(0, 8)
                def _(q):
                    nv = norm_v[pl.ds(q * 16, 16)]
                    for l in range(16):
                        sc = nv[l]
                        for f in range(nvec):
                            fs = pl.ds(f * 16, 16)
                            rows_v[q * 16 + l, fs] = rows_v[q * 16 + l, fs] * sc

                pltpu.sync_copy(rows_v, agg_sp.at[dst_v.at[g]], add=True)

        plsc.subcore_barrier()

        @pl.loop(0, 5)
        def _(i):
            off = base_n + i * 128
            pltpu.sync_copy(agg_sp.at[pl.ds(off, 128)], rows_v)
            pltpu.sync_copy(rows_v, out_hbm.at[c, pl.ds(off, 128)])

    return body_in_place if in_place else body


def _sc_calls():
    mesh = plsc.VectorSubcoreMesh(core_axis_name="c", subcore_axis_name="s")
    deg_call = pl.kernel(
        _deg_body,
        out_type=jax.ShapeDtypeStruct((NCORE, NPAD, 16), _f32),
        mesh=mesh,
        scratch_types=[
            pltpu.VMEM((40, 128), _i32),
            pltpu.VMEM((40, 128), _f32),
            pltpu.VMEM((128, 16), _f32),
            pltpu.VMEM((16, 16), _f32),
            pltpu.VMEM((128, 16), _f32),
            pltpu.VMEM_SHARED((NPAD, 16), _f32),
        ],
    )

    agg1_call = pl.kernel(
        _make_agg_body(128, feature_split=True),
        out_type=jax.ShapeDtypeStruct((NCORE, NPAD, 128), _f32),
        mesh=mesh,
        scratch_types=[
            pltpu.VMEM((8, 128), _i32),
            pltpu.VMEM((8, 128), _i32),
            pltpu.VMEM((8, 128), _f32),
            pltpu.VMEM((128,), _i32),
            pltpu.VMEM((128, 128), _f32),
            pltpu.VMEM((128,), _f32),
            pltpu.VMEM((128,), _f32),
            pltpu.VMEM((128,), _f32),
            pltpu.VMEM((16, 128), _f32),
            pltpu.VMEM_SHARED((NPAD, 128), _f32),
        ],
    )
    agg2_call = pl.kernel(
        _make_agg_body(32, feature_split=False),
        out_type=jax.ShapeDtypeStruct((NCORE, NPAD, 32), _f32),
        mesh=mesh,
        scratch_types=[
            pltpu.VMEM((8, 128), _i32),
            pltpu.VMEM((8, 128), _i32),
            pltpu.VMEM((8, 128), _f32),
            pltpu.VMEM((128, 128), _f32),
            pltpu.VMEM((128, 32), _f32),
            pltpu.VMEM((128,), _f32),
            pltpu.VMEM((128,), _f32),
            pltpu.VMEM((128,), _f32),
            pltpu.VMEM((16, 32), _f32),
            pltpu.VMEM_SHARED((NPAD, 32), _f32),
        ],
    )
    return deg_call, agg1_call, agg2_call


def _tc1_body(dp_ref, dinv_ref, d2_ref):
    deg = dp_ref[0] + dp_ref[1] + 1.0
    dv = lax.rsqrt(deg)
    dinv_ref[...] = dv
    d2_ref[...] = dv * dv


def _tc2_body(s1_ref, d2_ref, x_ref, w1_ref, b1_ref, w2_ref, z_ref):
    p1 = s1_ref[...] + d2_ref[...] * x_ref[...]
    h = jnp.maximum(
        jnp.dot(p1, w1_ref[...], preferred_element_type=_f32) + b1_ref[...], 0.0
    )
    z_ref[...] = jnp.dot(h, w2_ref[...], preferred_element_type=_f32)


def _tc3_body(s2_ref, z_ref, d2_ref, b2_ref, out_ref):
    logits = s2_ref[0] + s2_ref[1] + d2_ref[...] * z_ref[...] + b2_ref[...]
    m = jnp.max(logits, axis=1, keepdims=True)
    sh = logits - m
    out_ref[...] = sh - jnp.log(jnp.sum(jnp.exp(sh), axis=1, keepdims=True))


def kernel(x, edge_index, edge_weight, W1, b1, W2, b2):
    src = edge_index[0]
    dst = edge_index[1]
    pad_i = jnp.zeros((EP - E,), _i32)
    pad_f = jnp.zeros((EP - E,), _f32)
    src_p = jnp.concatenate([src, pad_i]).reshape(EROWS, 128)
    dst_p = jnp.concatenate([dst, pad_i]).reshape(EROWS, 128)
    ew_p = jnp.concatenate([edge_weight, pad_f]).reshape(EROWS, 128)

    deg_call, agg1_call, agg2_call = _sc_calls()

    deg16 = deg_call(dst_p, ew_p)
    deg_parts = deg16[:, :, 0].reshape(NCORE, 80, 128)

    dinv80, d2_80 = pl.pallas_call(
        _tc1_body,
        out_shape=(
            jax.ShapeDtypeStruct((80, 128), _f32),
            jax.ShapeDtypeStruct((80, 128), _f32),
        ),
    )(deg_parts)
    dinv_flat = dinv80.reshape(NPAD)
    d2col = d2_80.reshape(NPAD)[:N].reshape(N, 1)

    x_both = jnp.concatenate([x[:, :128], x[:, 128:]], axis=0)
    s1 = agg1_call(x_both, src_p, dst_p, ew_p, dinv_flat)
    s1cat = jnp.concatenate([s1[0, :N], s1[1, :N]], axis=1)

    blk = 1000
    grid = N // blk
    z = pl.pallas_call(
        _tc2_body,
        grid=(grid,),
        in_specs=[
            pl.BlockSpec((blk, NFEAT), lambda i: (i, 0)),
            pl.BlockSpec((blk, 1), lambda i: (i, 0)),
            pl.BlockSpec((blk, NFEAT), lambda i: (i, 0)),
            pl.BlockSpec((NFEAT, NHID), lambda i: (0, 0)),
            pl.BlockSpec((1, NHID), lambda i: (0, 0)),
            pl.BlockSpec((NHID, NCLASS), lambda i: (0, 0)),
        ],
        out_specs=pl.BlockSpec((blk, NCLASS), lambda i: (i, 0)),
        out_shape=jax.ShapeDtypeStruct((N, NCLASS), _f32),
    )(s1cat, d2col, x, W1, b1.reshape(1, NHID), W2)

    z_p = jnp.pad(z, ((0, NPAD - N), (0, 128 - NCLASS)))
    s2 = agg2_call(z_p, src_p, dst_p, ew_p, dinv_flat)

    out = pl.pallas_call(
        _tc3_body,
        out_shape=jax.ShapeDtypeStruct((N, NCLASS), _f32),
    )(s2[:, :N], z, d2col, b2.reshape(1, NCLASS))
    return out


# speedup vs baseline: 6.5741x; 1.0659x over previous
import jax
import jax.numpy as jnp
from jax import lax
from jax.experimental import pallas as pl
from jax.experimental.pallas import tpu as pltpu
from jax.experimental.pallas import tpu_sc as plsc

N = 10000
E = 160000
NFEAT = 256
NHID = 512
NCLASS = 32

NPAD = 10240
EP = 163840
EROWS = EP // 128
NSUB = 16
NCORE = 2

_f32 = jnp.float32
_i32 = jnp.int32


def _deg_body(dst_hbm, ew_hbm, out_hbm, dst_v, ew_v, rows_v, zbuf, bounce, deg_sp):
    c = lax.axis_index("c")
    s = lax.axis_index("s")
    wid = c * NSUB + s
    rpt = EROWS // (NCORE * NSUB)
    base_n = s * (NPAD // NSUB)

    zv = jnp.zeros((16,), _f32)
    for r in range(16):
        zbuf[r, :] = zv

    @pl.loop(0, (NPAD // NSUB) // 16)
    def _(i):
        pltpu.sync_copy(zbuf, deg_sp.at[pl.ds(base_n + i * 16, 16)])

    pltpu.sync_copy(dst_hbm.at[pl.ds(wid * rpt, rpt)], dst_v)
    pltpu.sync_copy(ew_hbm.at[pl.ds(wid * rpt, rpt)], ew_v)
    plsc.subcore_barrier()

    @pl.loop(0, rpt)
    def _(g):
        @pl.loop(0, 8)
        def _(q):
            wv = ew_v[g, pl.ds(q * 16, 16)]
            for l in range(16):
                rows_v[q * 16 + l, :] = jnp.full((16,), wv[l], _f32)

        pltpu.sync_copy(rows_v, deg_sp.at[dst_v.at[g]], add=True)

    plsc.subcore_barrier()

    @pl.loop(0, 5)
    def _(i):
        off = base_n + i * 128
        pltpu.sync_copy(deg_sp.at[pl.ds(off, 128)], bounce)
        pltpu.sync_copy(bounce, out_hbm.at[c, pl.ds(off, 128)])


def _agg1_body(t_hbm, src_hbm, dst_hbm, ew_hbm, dinv_hbm, out_hbm,
               src_v, dst_v, ew_v, idxa0, idxa1, rows0, rows1,
               dsrc0, dsrc1, ddst0, ddst1, norm_v, zbuf,
               semr0, semr1, sems0, sems1, semd0, semd1, agg_sp):
    c = lax.axis_index("c")
    s = lax.axis_index("s")
    rpt = EROWS // NSUB
    row0 = s * rpt
    base_n = s * (NPAD // NSUB)
    coff = c * N

    slots = (
        (idxa0, rows0, dsrc0, ddst0, semr0, sems0, semd0),
        (idxa1, rows1, dsrc1, ddst1, semr1, sems1, semd1),
    )

    zv = jnp.zeros((16,), _f32)
    for r in range(16):
        for f in range(8):
            zbuf[r, pl.ds(f * 16, 16)] = zv

    @pl.loop(0, (NPAD // NSUB) // 16)
    def _(i):
        pltpu.sync_copy(zbuf, agg_sp.at[pl.ds(base_n + i * 16, 16)])

    plsc.subcore_barrier()

    def fire(j):
        idxa, rows, dsrc, ddst, semr, sems, semd = slots[j % 2]
        for q in range(8):
            sl = pl.ds(q * 16, 16)
            idxa[sl] = src_v[j, sl] + coff
        dr = pltpu.async_copy(t_hbm.at[idxa], rows, semr)
        ds_ = pltpu.async_copy(dinv_hbm.at[src_v.at[j]], dsrc, sems)
        dd = pltpu.async_copy(dinv_hbm.at[dst_v.at[j]], ddst, semd)
        return (dr, ds_, dd)

    @pl.loop(0, rpt // 8)
    def _(b):
        pltpu.sync_copy(src_hbm.at[pl.ds(row0 + b * 8, 8)], src_v)
        pltpu.sync_copy(dst_hbm.at[pl.ds(row0 + b * 8, 8)], dst_v)
        pltpu.sync_copy(ew_hbm.at[pl.ds(row0 + b * 8, 8)], ew_v)

        descs = fire(0)
        for j in range(8):
            idxa, rows, dsrc, ddst, semr, sems, semd = slots[j % 2]
            for d in descs:
                d.wait()
            if j < 7:
                descs = fire(j + 1)

            for q in range(8):
                sl = pl.ds(q * 16, 16)
                norm_v[sl] = dsrc[sl] * ew_v[j, sl] * ddst[sl]

            @pl.loop(0, 8)
            def _(q):
                nv = norm_v[pl.ds(q * 16, 16)]
                for l in range(16):
                    sc = nv[l]
                    for f in range(8):
                        fs = pl.ds(f * 16, 16)
                        rows[q * 16 + l, fs] = rows[q * 16 + l, fs] * sc

            pltpu.sync_copy(rows, agg_sp.at[dst_v.at[j]], add=True)

    plsc.subcore_barrier()

    @pl.loop(0, 5)
    def _(i):
        off = base_n + i * 128
        pltpu.sync_copy(agg_sp.at[pl.ds(off, 128)], rows0)
        pltpu.sync_copy(rows0, out_hbm.at[c, pl.ds(off, 128)])


def _agg2_body(z_hbm, src_hbm, dst_hbm, ew_hbm, dinv_hbm, out_hbm,
               src_v, dst_v, ew_v, rows_v, out_rows_v,
               dsrc_v, ddst_v, norm_v, zbuf, agg_sp):
    c = lax.axis_index("c")
    s = lax.axis_index("s")
    rpt = EROWS // (NCORE * NSUB)
    row0 = (c * NSUB + s) * rpt
    base_n = s * (NPAD // NSUB)

    zv = jnp.zeros((16,), _f32)
    for r in range(16):
        for f in range(2):
            zbuf[r, pl.ds(f * 16, 16)] = zv

    @pl.loop(0, (NPAD // NSUB) // 16)
    def _(i):
        pltpu.sync_copy(zbuf, agg_sp.at[pl.ds(base_n + i * 16, 16)])

    plsc.subcore_barrier()

    @pl.loop(0, rpt // 8)
    def _(b):
        pltpu.sync_copy(src_hbm.at[pl.ds(row0 + b * 8, 8)], src_v)
        pltpu.sync_copy(dst_hbm.at[pl.ds(row0 + b * 8, 8)], dst_v)
        pltpu.sync_copy(ew_hbm.at[pl.ds(row0 + b * 8, 8)], ew_v)

        @pl.loop(0, 8)
        def _(g):
            pltpu.sync_copy(z_hbm.at[src_v.at[g]], rows_v)
            pltpu.sync_copy(dinv_hbm.at[src_v.at[g]], dsrc_v)
            pltpu.sync_copy(dinv_hbm.at[dst_v.at[g]], ddst_v)

            for q in range(8):
                sl = pl.ds(q * 16, 16)
                norm_v[sl] = dsrc_v[sl] * ew_v[g, sl] * ddst_v[sl]

            @pl.loop(0, 8)
            def _(q):
                nv = norm_v[pl.ds(q * 16, 16)]
                for l in range(16):
                    sc = nv[l]
                    for f in range(2):
                        fs = pl.ds(f * 16, 16)
                        out_rows_v[q * 16 + l, fs] = rows_v[q * 16 + l, fs] * sc

            pltpu.sync_copy(out_rows_v, agg_sp.at[dst_v.at[g]], add=True)

    plsc.subcore_barrier()

    @pl.loop(0, 5)
    def _(i):
        off = base_n + i * 128
        pltpu.sync_copy(agg_sp.at[pl.ds(off, 128)], out_rows_v)
        pltpu.sync_copy(out_rows_v, out_hbm.at[c, pl.ds(off, 128)])


def _sc_calls():
    mesh = plsc.VectorSubcoreMesh(core_axis_name="c", subcore_axis_name="s")
    deg_call = pl.kernel(
        _deg_body,
        out_type=jax.ShapeDtypeStruct((NCORE, NPAD, 16), _f32),
        mesh=mesh,
        scratch_types=[
            pltpu.VMEM((40, 128), _i32),
            pltpu.VMEM((40, 128), _f32),
            pltpu.VMEM((128, 16), _f32),
            pltpu.VMEM((16, 16), _f32),
            pltpu.VMEM((128, 16), _f32),
            pltpu.VMEM_SHARED((NPAD, 16), _f32),
        ],
    )
    agg1_call = pl.kernel(
        _agg1_body,
        out_type=jax.ShapeDtypeStruct((NCORE, NPAD, 128), _f32),
        mesh=mesh,
        scratch_types=[
            pltpu.VMEM((8, 128), _i32),
            pltpu.VMEM((8, 128), _i32),
            pltpu.VMEM((8, 128), _f32),
            pltpu.VMEM((128,), _i32),
            pltpu.VMEM((128,), _i32),
            pltpu.VMEM((128, 128), _f32),
            pltpu.VMEM((128, 128), _f32),
            pltpu.VMEM((128,), _f32),
            pltpu.VMEM((128,), _f32),
            pltpu.VMEM((128,), _f32),
            pltpu.VMEM((128,), _f32),
            pltpu.VMEM((128,), _f32),
            pltpu.VMEM((16, 128), _f32),
            pltpu.SemaphoreType.DMA,
            pltpu.SemaphoreType.DMA,
            pltpu.SemaphoreType.DMA,
            pltpu.SemaphoreType.DMA,
            pltpu.SemaphoreType.DMA,
            pltpu.SemaphoreType.DMA,
            pltpu.VMEM_SHARED((NPAD, 128), _f32),
        ],
    )
    agg2_call = pl.kernel(
        _agg2_body,
        out_type=jax.ShapeDtypeStruct((NCORE, NPAD, 32), _f32),
        mesh=mesh,
        scratch_types=[
            pltpu.VMEM((8, 128), _i32),
            pltpu.VMEM((8, 128), _i32),
            pltpu.VMEM((8, 128), _f32),
            pltpu.VMEM((128, 128), _f32),
            pltpu.VMEM((128, 32), _f32),
            pltpu.VMEM((128,), _f32),
            pltpu.VMEM((128,), _f32),
            pltpu.VMEM((128,), _f32),
            pltpu.VMEM((16, 32), _f32),
            pltpu.VMEM_SHARED((NPAD, 32), _f32),
        ],
    )
    return deg_call, agg1_call, agg2_call


def _tc1_body(dp_ref, dinv_ref, d2_ref):
    deg = dp_ref[0] + dp_ref[1] + 1.0
    dv = lax.rsqrt(deg)
    dinv_ref[...] = dv
    d2_ref[...] = dv * dv


def _tc2_body(s1_ref, d2_ref, x_ref, w1_ref, b1_ref, w2_ref, z_ref):
    p1 = s1_ref[...] + d2_ref[...] * x_ref[...]
    h = jnp.maximum(
        jnp.dot(p1, w1_ref[...], preferred_element_type=_f32) + b1_ref[...], 0.0
    )
    z_ref[...] = jnp.dot(h, w2_ref[...], preferred_element_type=_f32)


def _tc3_body(s2_ref, z_ref, d2_ref, b2_ref, out_ref):
    logits = s2_ref[0] + s2_ref[1] + d2_ref[...] * z_ref[...] + b2_ref[...]
    m = jnp.max(logits, axis=1, keepdims=True)
    sh = logits - m
    out_ref[...] = sh - jnp.log(jnp.sum(jnp.exp(sh), axis=1, keepdims=True))


def kernel(x, edge_index, edge_weight, W1, b1, W2, b2):
    src = edge_index[0]
    dst = edge_index[1]
    pad_i = jnp.zeros((EP - E,), _i32)
    pad_f = jnp.zeros((EP - E,), _f32)
    src_p = jnp.concatenate([src, pad_i]).reshape(EROWS, 128)
    dst_p = jnp.concatenate([dst, pad_i]).reshape(EROWS, 128)
    ew_p = jnp.concatenate([edge_weight, pad_f]).reshape(EROWS, 128)

    deg_call, agg1_call, agg2_call = _sc_calls()

    deg16 = deg_call(dst_p, ew_p)
    deg_parts = deg16[:, :, 0].reshape(NCORE, 80, 128)

    dinv80, d2_80 = pl.pallas_call(
        _tc1_body,
        out_shape=(
            jax.ShapeDtypeStruct((80, 128), _f32),
            jax.ShapeDtypeStruct((80, 128), _f32),
        ),
    )(deg_parts)
    dinv_flat = dinv80.reshape(NPAD)
    d2col = d2_80.reshape(NPAD)[:N].reshape(N, 1)

    x_both = jnp.concatenate([x[:, :128], x[:, 128:]], axis=0)
    s1 = agg1_call(x_both, src_p, dst_p, ew_p, dinv_flat)
    s1cat = jnp.concatenate([s1[0, :N], s1[1, :N]], axis=1)

    blk = 1000
    grid = N // blk
    z = pl.pallas_call(
        _tc2_body,
        grid=(grid,),
        in_specs=[
            pl.BlockSpec((blk, NFEAT), lambda i: (i, 0)),
            pl.BlockSpec((blk, 1), lambda i: (i, 0)),
            pl.BlockSpec((blk, NFEAT), lambda i: (i, 0)),
            pl.BlockSpec((NFEAT, NHID), lambda i: (0, 0)),
            pl.BlockSpec((1, NHID), lambda i: (0, 0)),
            pl.BlockSpec((NHID, NCLASS), lambda i: (0, 0)),
        ],
        out_specs=pl.BlockSpec((blk, NCLASS), lambda i: (i, 0)),
        out_shape=jax.ShapeDtypeStruct((N, NCLASS), _f32),
    )(s1cat, d2col, x, W1, b1.reshape(1, NHID), W2)

    z_p = jnp.pad(z, ((0, NPAD - N), (0, 128 - NCLASS)))
    s2 = agg2_call(z_p, src_p, dst_p, ew_p, dinv_flat)

    out = pl.pallas_call(
        _tc3_body,
        out_shape=jax.ShapeDtypeStruct((N, NCLASS), _f32),
    )(s2[:, :N], z, d2col, b2.reshape(1, NCLASS))
    return out
